# baseline (speedup 1.0000x reference)
import functools

import jax
import jax.numpy as jnp
from jax import lax
from jax.experimental import pallas as pl
from jax.experimental.pallas import tpu as pltpu
from jax.experimental.pallas import tpu_sc as plsc

B = 4096
T = 200
D = 64
SCALE = 8.0

_NC = 2
_NS = 16
_NW = _NC * _NS
_SEQ_PER_W = B // _NW
_HALF = T // 2

_mesh = plsc.VectorSubcoreMesh(core_axis_name="c", subcore_axis_name="s")


@functools.partial(
    pl.kernel,
    mesh=_mesh,
    compiler_params=pltpu.CompilerParams(use_tc_tiling_on_sc=False),
    out_type=jax.ShapeDtypeStruct((B * T, D), jnp.float32),
    scratch_types=[
        pltpu.VMEM((2, _HALF), jnp.int32),
        pltpu.VMEM((T, D), jnp.float32),
        pltpu.VMEM((T, D), jnp.float32),
        pltpu.SemaphoreType.DMA,
    ],
)
def _emb_kernel(x_hbm, ww_hbm, wp_hbm, out_hbm, idx_v, g_v, pos_v, sem):
    wid = lax.axis_index("s") * _NC + lax.axis_index("c")

    pltpu.sync_copy(wp_hbm.at[pl.ds(0, T)], pos_v)

    def body_b(i, carry):
        b = wid * _SEQ_PER_W + i
        pltpu.sync_copy(x_hbm.at[pl.ds(2 * b, 2)], idx_v)
        cp0 = pltpu.async_copy(ww_hbm.at[idx_v.at[0]], g_v.at[pl.ds(0, _HALF)], sem)
        cp1 = pltpu.async_copy(ww_hbm.at[idx_v.at[1]], g_v.at[pl.ds(_HALF, _HALF)], sem)
        cp0.wait()
        cp1.wait()

        def body_r(r, carry_r):
            for c in range(D // 16):
                w = g_v[r, pl.ds(c * 16, 16)]
                p = pos_v[r, pl.ds(c * 16, 16)]
                g_v[r, pl.ds(c * 16, 16)] = w * SCALE + p
            return carry_r

        lax.fori_loop(0, T, body_r, 0)
        pltpu.sync_copy(g_v, out_hbm.at[pl.ds(b * T, T)])
        return carry

    lax.fori_loop(0, _SEQ_PER_W, body_b, 0)


def kernel(x, W_word, W_pos):
    x2 = x.reshape(2 * B, _HALF).astype(jnp.int32)
    out = _emb_kernel(x2, W_word, W_pos)
    return out.reshape(B, T, D)
